# baseline (speedup 1.0000x reference)
import functools

import jax
import jax.numpy as jnp
from jax import lax
from jax.experimental import pallas as pl
from jax.experimental.pallas import tpu as pltpu
from jax.experimental.pallas import tpu_sc as plsc

N_TOK = 4096
D = 768
E = 8
BLKR = 1024
NBR = N_TOK // BLKR
BLKP = 1024
NBP = N_TOK // BLKP
BLKM = 256
PADTOT = 8192 + E * BLKM
NBM = PADTOT // BLKM
NW = 32
TPW = N_TOK // NW
CCH = 32


def _cv2(v):
    m = jnp.mean(v)
    var = jnp.mean((v - m) ** 2)
    return var / (m * m + 1e-10)


def _router_body(x_ref, wg_ref, e0_ref, e1_ref, g0_ref, g1_ref, loss_ref,
                 imp_ref, load_ref):
    i = pl.program_id(0)
    logits = jnp.dot(x_ref[...], wg_ref[...],
                     preferred_element_type=jnp.float32)
    e_iota = lax.broadcasted_iota(jnp.int32, logits.shape, 1)
    v1 = jnp.max(logits, axis=1, keepdims=True)
    idx1 = jnp.min(jnp.where(logits == v1, e_iota, E), axis=1, keepdims=True)
    masked = jnp.where(e_iota == idx1, -jnp.inf, logits)
    v2 = jnp.max(masked, axis=1, keepdims=True)
    idx2 = jnp.min(jnp.where(masked == v2, e_iota, E), axis=1, keepdims=True)

    e2 = jnp.exp(v2 - v1)
    denom = 1.0 + e2
    g1 = 1.0 / denom
    g2 = e2 / denom

    e0_ref[...] = idx1
    e1_ref[...] = idx2
    g0_ref[...] = jnp.broadcast_to(g1, (BLKR, 16))
    g1_ref[...] = jnp.broadcast_to(g2, (BLKR, 16))

    gates = (jnp.where(e_iota == idx1, g1, 0.0)
             + jnp.where(e_iota == idx2, g2, 0.0))
    imp_blk = jnp.sum(gates, axis=0, keepdims=True)
    load_blk = jnp.sum((gates > 0.0).astype(jnp.float32), axis=0,
                       keepdims=True)

    @pl.when(i == 0)
    def _init():
        imp_ref[...] = jnp.zeros_like(imp_ref)
        load_ref[...] = jnp.zeros_like(load_ref)

    imp_ref[...] += imp_blk
    load_ref[...] += load_blk

    @pl.when(i == pl.num_programs(0) - 1)
    def _fin():
        loss = _cv2(imp_ref[...]) + _cv2(load_ref[...])
        loss_ref[...] = jnp.broadcast_to(loss, (1, 1))


def _router(x, Wg):
    return pl.pallas_call(
        _router_body,
        grid=(NBR,),
        in_specs=[
            pl.BlockSpec((BLKR, D), lambda i: (i, 0)),
            pl.BlockSpec((D, E), lambda i: (0, 0)),
        ],
        out_specs=[
            pl.BlockSpec((BLKR, 1), lambda i: (i, 0)),
            pl.BlockSpec((BLKR, 1), lambda i: (i, 0)),
            pl.BlockSpec((BLKR, 16), lambda i: (i, 0)),
            pl.BlockSpec((BLKR, 16), lambda i: (i, 0)),
            pl.BlockSpec((1, 1), lambda i: (0, 0)),
        ],
        out_shape=[
            jax.ShapeDtypeStruct((N_TOK, 1), jnp.int32),
            jax.ShapeDtypeStruct((N_TOK, 1), jnp.int32),
            jax.ShapeDtypeStruct((N_TOK, 16), jnp.float32),
            jax.ShapeDtypeStruct((N_TOK, 16), jnp.float32),
            jax.ShapeDtypeStruct((1, 1), jnp.float32),
        ],
        scratch_shapes=[
            pltpu.VMEM((1, E), jnp.float32),
            pltpu.VMEM((1, E), jnp.float32),
        ],
    )(x, Wg)


def _perm_body(e0_ref, e1_ref, d0_ref, d1_ref, be_ref,
               cnt_ref, offs_ref, lst_ref):
    i = pl.program_id(0)
    phase = i // (2 * NBP)
    k = (i // NBP) % 2
    blk = i % NBP

    c0 = e0_ref[pl.ds(blk * BLKP, BLKP), :]
    c1 = e1_ref[pl.ds(blk * BLKP, BLKP), :]
    e_col = jnp.where(k == 0, c0, c1)
    oh = (e_col == lax.broadcasted_iota(jnp.int32, (BLKP, E), 1)
          ).astype(jnp.float32)

    @pl.when(i == 0)
    def _init():
        cnt_ref[...] = jnp.zeros_like(cnt_ref)

    @pl.when(phase == 0)
    def _count():
        cnt_ref[...] += jnp.sum(oh, axis=0, keepdims=True)

    @pl.when(i == 2 * NBP)
    def _boundary():
        cnt = cnt_ref[...]
        al = jnp.floor((cnt + (BLKM - 1.0)) / BLKM) * BLKM
        sub = lax.broadcasted_iota(jnp.int32, (E, E), 0)
        lane = lax.broadcasted_iota(jnp.int32, (E, E), 1)
        eye = (sub == lane).astype(jnp.float32)
        tstrict = (sub < lane).astype(jnp.float32)
        al_col = jnp.sum(eye * al, axis=1, keepdims=True)
        offs = jnp.sum(al_col * tstrict, axis=0, keepdims=True)
        offs_ref[...] = offs
        ends_col = jnp.sum(eye * (offs + al), axis=1, keepdims=True)
        j256 = (lax.broadcasted_iota(jnp.int32, (E, 128), 1)
                * BLKM).astype(jnp.float32)
        be = jnp.sum((ends_col <= j256).astype(jnp.float32), axis=0,
                     keepdims=True)
        be_ref[...] = jnp.minimum(be, E - 1.0).astype(jnp.int32)
        cnt_ref[...] = jnp.zeros_like(cnt_ref)
        r = lax.broadcasted_iota(jnp.int32, (BLKP, BLKP), 0)
        c = lax.broadcasted_iota(jnp.int32, (BLKP, BLKP), 1)
        lst_ref[...] = (c < r).astype(jnp.bfloat16)

    @pl.when(phase == 1)
    def _rank():
        carry = cnt_ref[...]
        rexc = jnp.dot(lst_ref[...], oh.astype(jnp.bfloat16),
                       preferred_element_type=jnp.float32)
        dest = jnp.sum(oh * (rexc + carry + offs_ref[...]), axis=1,
                       keepdims=True).astype(jnp.int32)
        d0_ref[...] = dest
        d1_ref[...] = dest
        cnt_ref[...] += jnp.sum(oh, axis=0, keepdims=True)


def _perm(e0, e1):
    park = NBP

    def _d0_map(i):
        ph1k0 = jnp.logical_and(i >= 2 * NBP, i < 3 * NBP)
        return (jnp.where(ph1k0, i - 2 * NBP, park), 0)

    def _d1_map(i):
        ph1k1 = i >= 3 * NBP
        return (jnp.where(ph1k1, i - 3 * NBP, park), 0)

    d0, d1, be = pl.pallas_call(
        _perm_body,
        grid=(4 * NBP,),
        in_specs=[
            pl.BlockSpec((N_TOK, 1), lambda i: (0, 0)),
            pl.BlockSpec((N_TOK, 1), lambda i: (0, 0)),
        ],
        out_specs=[
            pl.BlockSpec((BLKP, 1), _d0_map),
            pl.BlockSpec((BLKP, 1), _d1_map),
            pl.BlockSpec((1, 128), lambda i: (0, 0)),
        ],
        out_shape=[
            jax.ShapeDtypeStruct((N_TOK + BLKP, 1), jnp.int32),
            jax.ShapeDtypeStruct((N_TOK + BLKP, 1), jnp.int32),
            jax.ShapeDtypeStruct((1, 128), jnp.int32),
        ],
        scratch_shapes=[
            pltpu.VMEM((1, E), jnp.float32),
            pltpu.VMEM((1, E), jnp.float32),
            pltpu.VMEM((BLKP, BLKP), jnp.bfloat16),
        ],
    )(e0, e1)
    return d0[:N_TOK], d1[:N_TOK], be


def _dispatch(x, d0, d1):
    mesh = plsc.VectorSubcoreMesh(core_axis_name="c", subcore_axis_name="s")

    @functools.partial(
        pl.kernel, mesh=mesh,
        out_type=jax.ShapeDtypeStruct((PADTOT, D), jnp.float32),
        scratch_types=[
            pltpu.VMEM((TPW,), jnp.int32),
            pltpu.VMEM((TPW,), jnp.int32),
            pltpu.VMEM((TPW, D), jnp.float32),
            pltpu.SemaphoreType.DMA,
        ],
    )
    def disp(x_hbm, d0_hbm, d1_hbm, xs_hbm, i0_v, i1_v, rows_v, sem):
        wid = lax.axis_index("s") * 2 + lax.axis_index("c")
        base = wid * TPW
        pltpu.sync_copy(d0_hbm.at[pl.ds(base, TPW)], i0_v)
        pltpu.sync_copy(d1_hbm.at[pl.ds(base, TPW)], i1_v)
        pltpu.sync_copy(x_hbm.at[pl.ds(base, TPW)], rows_v)
        pltpu.async_copy(rows_v, xs_hbm.at[i0_v], sem).wait()
        pltpu.async_copy(rows_v, xs_hbm.at[i1_v], sem).wait()

    return disp(x, d0.reshape(N_TOK), d1.reshape(N_TOK))


def _gmm_body(be_ref, xs_ref, wb_ref, b_ref, y_ref):
    i = pl.program_id(0)
    e = be_ref[0, i]
    xb = xs_ref[...].astype(jnp.bfloat16)
    y = jnp.dot(xb, wb_ref[pl.ds(e * D, D)],
                preferred_element_type=jnp.float32)
    y += b_ref[pl.ds(e, 1)]
    y_ref[...] = y


def _gmm(be, xs, wb, b):
    return pl.pallas_call(
        _gmm_body,
        grid=(NBM,),
        in_specs=[
            pl.BlockSpec(memory_space=pltpu.SMEM),
            pl.BlockSpec((BLKM, D), lambda i: (i, 0)),
            pl.BlockSpec((E * D, D), lambda i: (0, 0)),
            pl.BlockSpec((E, D), lambda i: (0, 0)),
        ],
        out_specs=pl.BlockSpec((BLKM, D), lambda i: (i, 0)),
        out_shape=jax.ShapeDtypeStruct((PADTOT, D), jnp.float32),
    )(be, xs, wb, b)


def _combine(y, d0, d1, g0, g1):
    mesh = plsc.VectorSubcoreMesh(core_axis_name="c", subcore_axis_name="s")
    n_ch = TPW // CCH

    @functools.partial(
        pl.kernel, mesh=mesh,
        out_type=jax.ShapeDtypeStruct((N_TOK, D), jnp.float32),
        scratch_types=[
            pltpu.VMEM((CCH,), jnp.int32),
            pltpu.VMEM((CCH,), jnp.int32),
            pltpu.VMEM((CCH, 16), jnp.float32),
            pltpu.VMEM((CCH, 16), jnp.float32),
            pltpu.VMEM((CCH, D), jnp.float32),
            pltpu.VMEM((CCH, D), jnp.float32),
            pltpu.VMEM((CCH, D), jnp.float32),
            pltpu.SemaphoreType.DMA,
        ],
    )
    def comb(y_hbm, d0_hbm, d1_hbm, g0_hbm, g1_hbm, out_hbm,
             i0_v, i1_v, g0_v, g1_v, ys0_v, ys1_v, orows_v, sem):
        wid = lax.axis_index("s") * 2 + lax.axis_index("c")
        base = wid * TPW
        for ch in range(n_ch):
            t0 = base + ch * CCH
            pltpu.sync_copy(d0_hbm.at[pl.ds(t0, CCH)], i0_v)
            pltpu.sync_copy(d1_hbm.at[pl.ds(t0, CCH)], i1_v)
            pltpu.sync_copy(g0_hbm.at[pl.ds(t0, CCH)], g0_v)
            pltpu.sync_copy(g1_hbm.at[pl.ds(t0, CCH)], g1_v)
            pltpu.async_copy(y_hbm.at[i0_v], ys0_v, sem).wait()
            pltpu.async_copy(y_hbm.at[i1_v], ys1_v, sem).wait()

            def body(t, _):
                ga = g0_v[t, :]
                gb = g1_v[t, :]
                for j in range(D // 16):
                    a = ys0_v[t, pl.ds(j * 16, 16)]
                    bv = ys1_v[t, pl.ds(j * 16, 16)]
                    orows_v[t, pl.ds(j * 16, 16)] = ga * a + gb * bv
                return 0

            lax.fori_loop(0, CCH, body, 0)
            pltpu.sync_copy(orows_v, out_hbm.at[pl.ds(t0, CCH)])

    return comb(y, d0.reshape(N_TOK), d1.reshape(N_TOK), g0, g1)


@jax.jit
def kernel(x, Wg, W, b):
    wb = W.astype(jnp.bfloat16).reshape(E * D, D)
    e0, e1, g0, g1, loss = _router(x, Wg)
    d0, d1, be = _perm(e0, e1)
    xs = _dispatch(x, d0, d1)
    y = _gmm(be, xs, wb, b)
    out = _combine(y, d0, d1, g0, g1)
    return out, loss.reshape(())


# speedup vs baseline: 1.0071x; 1.0071x over previous
import functools

import jax
import jax.numpy as jnp
from jax import lax
from jax.experimental import pallas as pl
from jax.experimental.pallas import tpu as pltpu
from jax.experimental.pallas import tpu_sc as plsc

N_TOK = 4096
D = 768
E = 8
BLKR = 1024
NBR = N_TOK // BLKR
BLKP = 1024
NBP = N_TOK // BLKP
BLKM = 256
PADTOT = 8192 + E * BLKM
NBM = PADTOT // BLKM
NW = 32
TPW = N_TOK // NW
CCH = 32


def _cv2(v):
    m = jnp.mean(v)
    var = jnp.mean((v - m) ** 2)
    return var / (m * m + 1e-10)


def _router_body(x_ref, wg_ref, e0_ref, e1_ref, g0_ref, g1_ref, loss_ref,
                 imp_ref, load_ref):
    i = pl.program_id(0)
    logits = jnp.dot(x_ref[...], wg_ref[...],
                     preferred_element_type=jnp.float32)
    e_iota = lax.broadcasted_iota(jnp.int32, logits.shape, 1)
    v1 = jnp.max(logits, axis=1, keepdims=True)
    idx1 = jnp.min(jnp.where(logits == v1, e_iota, E), axis=1, keepdims=True)
    masked = jnp.where(e_iota == idx1, -jnp.inf, logits)
    v2 = jnp.max(masked, axis=1, keepdims=True)
    idx2 = jnp.min(jnp.where(masked == v2, e_iota, E), axis=1, keepdims=True)

    e2 = jnp.exp(v2 - v1)
    denom = 1.0 + e2
    g1 = 1.0 / denom
    g2 = e2 / denom

    e0_ref[...] = idx1
    e1_ref[...] = idx2
    g0_ref[...] = jnp.broadcast_to(g1, (BLKR, 16))
    g1_ref[...] = jnp.broadcast_to(g2, (BLKR, 16))

    gates = (jnp.where(e_iota == idx1, g1, 0.0)
             + jnp.where(e_iota == idx2, g2, 0.0))
    imp_blk = jnp.sum(gates, axis=0, keepdims=True)
    load_blk = jnp.sum((gates > 0.0).astype(jnp.float32), axis=0,
                       keepdims=True)

    @pl.when(i == 0)
    def _init():
        imp_ref[...] = jnp.zeros_like(imp_ref)
        load_ref[...] = jnp.zeros_like(load_ref)

    imp_ref[...] += imp_blk
    load_ref[...] += load_blk

    @pl.when(i == pl.num_programs(0) - 1)
    def _fin():
        loss = _cv2(imp_ref[...]) + _cv2(load_ref[...])
        loss_ref[...] = jnp.broadcast_to(loss, (1, 1))


def _router(x, Wg):
    return pl.pallas_call(
        _router_body,
        grid=(NBR,),
        in_specs=[
            pl.BlockSpec((BLKR, D), lambda i: (i, 0)),
            pl.BlockSpec((D, E), lambda i: (0, 0)),
        ],
        out_specs=[
            pl.BlockSpec((BLKR, 1), lambda i: (i, 0)),
            pl.BlockSpec((BLKR, 1), lambda i: (i, 0)),
            pl.BlockSpec((BLKR, 16), lambda i: (i, 0)),
            pl.BlockSpec((BLKR, 16), lambda i: (i, 0)),
            pl.BlockSpec((1, 1), lambda i: (0, 0)),
        ],
        out_shape=[
            jax.ShapeDtypeStruct((N_TOK, 1), jnp.int32),
            jax.ShapeDtypeStruct((N_TOK, 1), jnp.int32),
            jax.ShapeDtypeStruct((N_TOK, 16), jnp.float32),
            jax.ShapeDtypeStruct((N_TOK, 16), jnp.float32),
            jax.ShapeDtypeStruct((1, 1), jnp.float32),
        ],
        scratch_shapes=[
            pltpu.VMEM((1, E), jnp.float32),
            pltpu.VMEM((1, E), jnp.float32),
        ],
    )(x, Wg)


def _perm_body(e0_ref, e1_ref, d0_ref, d1_ref, be_ref,
               cnt_ref, offs_ref, lst_ref):
    i = pl.program_id(0)
    phase = i // (2 * NBP)
    k = (i // NBP) % 2
    blk = i % NBP

    c0 = e0_ref[pl.ds(blk * BLKP, BLKP), :]
    c1 = e1_ref[pl.ds(blk * BLKP, BLKP), :]
    e_col = jnp.where(k == 0, c0, c1)
    oh = (e_col == lax.broadcasted_iota(jnp.int32, (BLKP, E), 1)
          ).astype(jnp.float32)

    @pl.when(i == 0)
    def _init():
        cnt_ref[...] = jnp.zeros_like(cnt_ref)

    @pl.when(phase == 0)
    def _count():
        cnt_ref[...] += jnp.sum(oh, axis=0, keepdims=True)

    @pl.when(i == 2 * NBP)
    def _boundary():
        cnt = cnt_ref[...]
        al = jnp.floor((cnt + (BLKM - 1.0)) / BLKM) * BLKM
        sub = lax.broadcasted_iota(jnp.int32, (E, E), 0)
        lane = lax.broadcasted_iota(jnp.int32, (E, E), 1)
        eye = (sub == lane).astype(jnp.float32)
        tstrict = (sub < lane).astype(jnp.float32)
        al_col = jnp.sum(eye * al, axis=1, keepdims=True)
        offs = jnp.sum(al_col * tstrict, axis=0, keepdims=True)
        offs_ref[...] = offs
        ends_col = jnp.sum(eye * (offs + al), axis=1, keepdims=True)
        j256 = (lax.broadcasted_iota(jnp.int32, (E, 128), 1)
                * BLKM).astype(jnp.float32)
        be = jnp.sum((ends_col <= j256).astype(jnp.float32), axis=0,
                     keepdims=True)
        be_ref[...] = jnp.minimum(be, E - 1.0).astype(jnp.int32)
        cnt_ref[...] = jnp.zeros_like(cnt_ref)
        r = lax.broadcasted_iota(jnp.int32, (BLKP, BLKP), 0)
        c = lax.broadcasted_iota(jnp.int32, (BLKP, BLKP), 1)
        lst_ref[...] = (c < r).astype(jnp.bfloat16)

    @pl.when(phase == 1)
    def _rank():
        carry = cnt_ref[...]
        rexc = jnp.dot(lst_ref[...], oh.astype(jnp.bfloat16),
                       preferred_element_type=jnp.float32)
        dest = jnp.sum(oh * (rexc + carry + offs_ref[...]), axis=1,
                       keepdims=True).astype(jnp.int32)
        d0_ref[...] = dest
        d1_ref[...] = dest
        cnt_ref[...] += jnp.sum(oh, axis=0, keepdims=True)


def _perm(e0, e1):
    park = NBP

    def _d0_map(i):
        ph1k0 = jnp.logical_and(i >= 2 * NBP, i < 3 * NBP)
        return (jnp.where(ph1k0, i - 2 * NBP, park), 0)

    def _d1_map(i):
        ph1k1 = i >= 3 * NBP
        return (jnp.where(ph1k1, i - 3 * NBP, park), 0)

    d0, d1, be = pl.pallas_call(
        _perm_body,
        grid=(4 * NBP,),
        in_specs=[
            pl.BlockSpec((N_TOK, 1), lambda i: (0, 0)),
            pl.BlockSpec((N_TOK, 1), lambda i: (0, 0)),
        ],
        out_specs=[
            pl.BlockSpec((BLKP, 1), _d0_map),
            pl.BlockSpec((BLKP, 1), _d1_map),
            pl.BlockSpec((1, 128), lambda i: (0, 0)),
        ],
        out_shape=[
            jax.ShapeDtypeStruct((N_TOK + BLKP, 1), jnp.int32),
            jax.ShapeDtypeStruct((N_TOK + BLKP, 1), jnp.int32),
            jax.ShapeDtypeStruct((1, 128), jnp.int32),
        ],
        scratch_shapes=[
            pltpu.VMEM((1, E), jnp.float32),
            pltpu.VMEM((1, E), jnp.float32),
            pltpu.VMEM((BLKP, BLKP), jnp.bfloat16),
        ],
    )(e0, e1)
    return d0[:N_TOK], d1[:N_TOK], be


def _dispatch(x, d0, d1):
    mesh = plsc.VectorSubcoreMesh(core_axis_name="c", subcore_axis_name="s")

    @functools.partial(
        pl.kernel, mesh=mesh,
        out_type=jax.ShapeDtypeStruct((PADTOT, D), jnp.float32),
        scratch_types=[
            pltpu.VMEM((TPW,), jnp.int32),
            pltpu.VMEM((TPW,), jnp.int32),
            pltpu.VMEM((TPW, D), jnp.float32),
            pltpu.SemaphoreType.DMA,
        ],
    )
    def disp(x_hbm, d0_hbm, d1_hbm, xs_hbm, i0_v, i1_v, rows_v, sem):
        wid = lax.axis_index("s") * 2 + lax.axis_index("c")
        base = wid * TPW
        pltpu.sync_copy(d0_hbm.at[pl.ds(base, TPW)], i0_v)
        pltpu.sync_copy(d1_hbm.at[pl.ds(base, TPW)], i1_v)
        pltpu.sync_copy(x_hbm.at[pl.ds(base, TPW)], rows_v)
        c1 = pltpu.async_copy(rows_v, xs_hbm.at[i0_v], sem)
        c2 = pltpu.async_copy(rows_v, xs_hbm.at[i1_v], sem)
        c1.wait()
        c2.wait()

    return disp(x, d0.reshape(N_TOK), d1.reshape(N_TOK))


def _gmm_body(be_ref, xs_ref, wb_ref, b_ref, y_ref):
    i = pl.program_id(0)
    e = be_ref[0, i]
    xb = xs_ref[...].astype(jnp.bfloat16)
    y = jnp.dot(xb, wb_ref[pl.ds(e * D, D)],
                preferred_element_type=jnp.float32)
    y += b_ref[pl.ds(e, 1)]
    y_ref[...] = y


def _gmm(be, xs, wb, b):
    return pl.pallas_call(
        _gmm_body,
        grid=(NBM,),
        in_specs=[
            pl.BlockSpec(memory_space=pltpu.SMEM),
            pl.BlockSpec((BLKM, D), lambda i: (i, 0)),
            pl.BlockSpec((E * D, D), lambda i: (0, 0)),
            pl.BlockSpec((E, D), lambda i: (0, 0)),
        ],
        out_specs=pl.BlockSpec((BLKM, D), lambda i: (i, 0)),
        out_shape=jax.ShapeDtypeStruct((PADTOT, D), jnp.float32),
    )(be, xs, wb, b)


def _combine(y, d0, d1, g0, g1):
    mesh = plsc.VectorSubcoreMesh(core_axis_name="c", subcore_axis_name="s")
    n_ch = TPW // CCH

    @functools.partial(
        pl.kernel, mesh=mesh,
        out_type=jax.ShapeDtypeStruct((N_TOK, D), jnp.float32),
        scratch_types=[
            pltpu.VMEM((TPW,), jnp.int32),
            pltpu.VMEM((TPW,), jnp.int32),
            pltpu.VMEM((TPW, 16), jnp.float32),
            pltpu.VMEM((TPW, 16), jnp.float32),
            pltpu.VMEM((2 * CCH,), jnp.int32),
            pltpu.VMEM((2 * CCH, D), jnp.float32),
            pltpu.VMEM((CCH, D), jnp.float32),
            pltpu.SemaphoreType.DMA,
        ],
    )
    def comb(y_hbm, d0_hbm, d1_hbm, g0_hbm, g1_hbm, out_hbm,
             i0_v, i1_v, ga_v, gb_v, icat_v, ys_v, orows_v, sem):
        wid = lax.axis_index("s") * 2 + lax.axis_index("c")
        base = wid * TPW
        pltpu.sync_copy(d0_hbm.at[pl.ds(base, TPW)], i0_v)
        pltpu.sync_copy(d1_hbm.at[pl.ds(base, TPW)], i1_v)
        pltpu.sync_copy(g0_hbm.at[pl.ds(base, TPW)], ga_v)
        pltpu.sync_copy(g1_hbm.at[pl.ds(base, TPW)], gb_v)
        for ch in range(n_ch):
            t0 = base + ch * CCH
            for h in range(CCH // 16):
                icat_v[pl.ds(h * 16, 16)] = i0_v[pl.ds(ch * CCH + h * 16, 16)]
                icat_v[pl.ds(CCH + h * 16, 16)] = (
                    i1_v[pl.ds(ch * CCH + h * 16, 16)])
            pltpu.async_copy(y_hbm.at[icat_v], ys_v, sem).wait()

            def body(t, _):
                ga = ga_v[ch * CCH + t, :]
                gb = gb_v[ch * CCH + t, :]
                for j in range(D // 16):
                    a = ys_v[t, pl.ds(j * 16, 16)]
                    bv = ys_v[t + CCH, pl.ds(j * 16, 16)]
                    orows_v[t, pl.ds(j * 16, 16)] = ga * a + gb * bv
                return 0

            lax.fori_loop(0, CCH, body, 0)
            pltpu.sync_copy(orows_v, out_hbm.at[pl.ds(t0, CCH)])

    return comb(y, d0.reshape(N_TOK), d1.reshape(N_TOK), g0, g1)


@jax.jit
def kernel(x, Wg, W, b):
    wb = W.astype(jnp.bfloat16).reshape(E * D, D)
    e0, e1, g0, g1, loss = _router(x, Wg)
    d0, d1, be = _perm(e0, e1)
    xs = _dispatch(x, d0, d1)
    y = _gmm(be, xs, wb, b)
    out = _combine(y, d0, d1, g0, g1)
    return out, loss.reshape(())


# speedup vs baseline: 2.2818x; 2.2657x over previous
import functools

import jax
import jax.numpy as jnp
from jax.experimental import pallas as pl
from jax.experimental.pallas import tpu as pltpu

N_TOK = 4096
D = 768
E = 8
BLK = 2048
NB = N_TOK // BLK


def _cv2(v):
    m = jnp.mean(v)
    var = jnp.mean((v - m) ** 2)
    return var / (m * m + 1e-10)


def _moe_body(x_ref, wg_ref, wb_ref, b_ref, out_ref, loss_ref, imp_ref, load_ref):
    i = pl.program_id(0)

    xf = x_ref[...]
    logits = jnp.dot(xf, wg_ref[...], preferred_element_type=jnp.float32)

    e_iota = jax.lax.broadcasted_iota(jnp.int32, logits.shape, 1)
    v1 = jnp.max(logits, axis=1, keepdims=True)
    idx1 = jnp.min(jnp.where(logits == v1, e_iota, E), axis=1, keepdims=True)
    masked = jnp.where(e_iota == idx1, -jnp.inf, logits)
    v2 = jnp.max(masked, axis=1, keepdims=True)
    idx2 = jnp.min(jnp.where(masked == v2, e_iota, E), axis=1, keepdims=True)

    e2 = jnp.exp(v2 - v1)
    denom = 1.0 + e2
    g1 = 1.0 / denom
    g2 = e2 / denom
    gates = (jnp.where(e_iota == idx1, g1, 0.0)
             + jnp.where(e_iota == idx2, g2, 0.0))

    acc = jnp.dot(gates, b_ref[...], preferred_element_type=jnp.float32)
    xb = xf.astype(jnp.bfloat16)
    for e in range(E):
        ye = jnp.dot(xb, wb_ref[pl.ds(e * D, D)],
                     preferred_element_type=jnp.float32)
        acc += gates[:, e:e + 1] * ye
    out_ref[...] = acc

    imp_blk = jnp.sum(gates, axis=0, keepdims=True)
    load_blk = jnp.sum((gates > 0.0).astype(jnp.float32), axis=0, keepdims=True)

    @pl.when(i == 0)
    def _init():
        imp_ref[...] = jnp.zeros_like(imp_ref)
        load_ref[...] = jnp.zeros_like(load_ref)

    imp_ref[...] += imp_blk
    load_ref[...] += load_blk

    @pl.when(i == pl.num_programs(0) - 1)
    def _fin():
        loss = _cv2(imp_ref[...]) + _cv2(load_ref[...])
        loss_ref[...] = jnp.broadcast_to(loss, (1, 1))


@jax.jit
def kernel(x, Wg, W, b):
    wb = W.astype(jnp.bfloat16).reshape(E * D, D)
    out, loss = pl.pallas_call(
        _moe_body,
        grid=(NB,),
        in_specs=[
            pl.BlockSpec((BLK, D), lambda i: (i, 0)),
            pl.BlockSpec((D, E), lambda i: (0, 0)),
            pl.BlockSpec((E * D, D), lambda i: (0, 0)),
            pl.BlockSpec((E, D), lambda i: (0, 0)),
        ],
        out_specs=[
            pl.BlockSpec((BLK, D), lambda i: (i, 0)),
            pl.BlockSpec((1, 1), lambda i: (0, 0)),
        ],
        out_shape=[
            jax.ShapeDtypeStruct((N_TOK, D), jnp.float32),
            jax.ShapeDtypeStruct((1, 1), jnp.float32),
        ],
        scratch_shapes=[
            pltpu.VMEM((1, E), jnp.float32),
            pltpu.VMEM((1, E), jnp.float32),
        ],
    )(x, Wg, wb, b)
    return out, loss.reshape(())


# speedup vs baseline: 2.6654x; 1.1681x over previous
import functools

import jax
import jax.numpy as jnp
from jax.experimental import pallas as pl
from jax.experimental.pallas import tpu as pltpu

N_TOK = 4096
D = 768
E = 8
BLK = 1024
NB = N_TOK // BLK


def _cv2(v):
    m = jnp.mean(v)
    var = jnp.mean((v - m) ** 2)
    return var / (m * m + 1e-10)


def _moe_body(x_ref, wg_ref, w_ref, b_ref, out_ref, loss_ref, imp_ref,
              load_ref, wb_ref):
    i = pl.program_id(0)

    @pl.when(i == 0)
    def _cast_w():
        wb_ref[...] = w_ref[...].astype(jnp.bfloat16)

    xf = x_ref[...]
    logits = jnp.dot(xf, wg_ref[...], preferred_element_type=jnp.float32)

    e_iota = jax.lax.broadcasted_iota(jnp.int32, logits.shape, 1)
    v1 = jnp.max(logits, axis=1, keepdims=True)
    idx1 = jnp.min(jnp.where(logits == v1, e_iota, E), axis=1, keepdims=True)
    masked = jnp.where(e_iota == idx1, -jnp.inf, logits)
    v2 = jnp.max(masked, axis=1, keepdims=True)
    idx2 = jnp.min(jnp.where(masked == v2, e_iota, E), axis=1, keepdims=True)

    e2 = jnp.exp(v2 - v1)
    denom = 1.0 + e2
    g1 = 1.0 / denom
    g2 = e2 / denom
    gates = (jnp.where(e_iota == idx1, g1, 0.0)
             + jnp.where(e_iota == idx2, g2, 0.0))

    acc = jnp.dot(gates, b_ref[...], preferred_element_type=jnp.float32)
    xb = xf.astype(jnp.bfloat16)
    for e in range(E):
        ye = jnp.dot(xb, wb_ref[pl.ds(e * D, D)],
                     preferred_element_type=jnp.float32)
        acc += gates[:, e:e + 1] * ye
    out_ref[...] = acc

    imp_blk = jnp.sum(gates, axis=0, keepdims=True)
    load_blk = jnp.sum((gates > 0.0).astype(jnp.float32), axis=0, keepdims=True)

    @pl.when(i == 0)
    def _init():
        imp_ref[...] = jnp.zeros_like(imp_ref)
        load_ref[...] = jnp.zeros_like(load_ref)

    imp_ref[...] += imp_blk
    load_ref[...] += load_blk

    @pl.when(i == pl.num_programs(0) - 1)
    def _fin():
        loss = _cv2(imp_ref[...]) + _cv2(load_ref[...])
        loss_ref[...] = jnp.broadcast_to(loss, (1, 1))


@jax.jit
def kernel(x, Wg, W, b):
    wb = W.reshape(E * D, D)
    out, loss = pl.pallas_call(
        _moe_body,
        grid=(NB,),
        in_specs=[
            pl.BlockSpec((BLK, D), lambda i: (i, 0)),
            pl.BlockSpec((D, E), lambda i: (0, 0)),
            pl.BlockSpec((E * D, D), lambda i: (0, 0)),
            pl.BlockSpec((E, D), lambda i: (0, 0)),
        ],
        out_specs=[
            pl.BlockSpec((BLK, D), lambda i: (i, 0)),
            pl.BlockSpec((1, 1), lambda i: (0, 0)),
        ],
        out_shape=[
            jax.ShapeDtypeStruct((N_TOK, D), jnp.float32),
            jax.ShapeDtypeStruct((1, 1), jnp.float32),
        ],
        scratch_shapes=[
            pltpu.VMEM((1, E), jnp.float32),
            pltpu.VMEM((1, E), jnp.float32),
            pltpu.VMEM((E * D, D), jnp.bfloat16),
        ],
    )(x, Wg, wb, b)
    return out, loss.reshape(())
